# Initial kernel scaffold; baseline (speedup 1.0000x reference)
import jax
import jax.numpy as jnp
from jax import lax
from jax.experimental import pallas as pl
from jax.experimental.pallas import tpu as pltpu
from jax.experimental.pallas import tpu_sc as plsc

N = 10000
E = 320000
D = 128

NC = 2
NS = 16
NW = NC * NS
EPT = E // NW
CH = 80
NCHUNK = EPT // CH
RPT = N // NS
DEGW = 16

ROWS_BLK = 2000


def _make_sc_segsum(with_deg: bool):
    mesh = plsc.VectorSubcoreMesh(core_axis_name="c", subcore_axis_name="s")
    out_type = [jax.ShapeDtypeStruct((NC, N, D), jnp.float32)]
    scratch = [
        pltpu.VMEM((CH,), jnp.int32),
        pltpu.VMEM((CH,), jnp.int32),
        pltpu.VMEM((CH, D), jnp.float32),
        pltpu.VMEM_SHARED((N, D), jnp.float32),
        pltpu.SemaphoreType.DMA,
    ]
    if with_deg:
        out_type.append(jax.ShapeDtypeStruct((NC, N, DEGW), jnp.float32))
        scratch += [
            pltpu.VMEM((CH, DEGW), jnp.float32),
            pltpu.VMEM_SHARED((N, DEGW), jnp.float32),
        ]

    def body(*refs):
        if with_deg:
            (g_hbm, src_hbm, dst_hbm, zrow_hbm, zdeg_hbm, ones_hbm,
             part_hbm, degp_hbm,
             sidx, didx, rows, acc_sh, sem, ones_v, deg_sh) = refs
        else:
            (g_hbm, src_hbm, dst_hbm, zrow_hbm,
             part_hbm,
             sidx, didx, rows, acc_sh, sem) = refs
        c = lax.axis_index("c")
        s = lax.axis_index("s")
        wid = c * NS + s

        pltpu.sync_copy(zrow_hbm, acc_sh.at[pl.ds(s * RPT, RPT)])
        if with_deg:
            pltpu.sync_copy(zdeg_hbm, deg_sh.at[pl.ds(s * RPT, RPT)])
            pltpu.sync_copy(ones_hbm, ones_v)
        plsc.subcore_barrier()

        base = pl.multiple_of(wid * EPT, 8)

        @pl.loop(0, NCHUNK)
        def _(j):
            off = pl.multiple_of(base + j * CH, 8)
            pltpu.sync_copy(src_hbm.at[pl.ds(off, CH)], sidx)
            pltpu.sync_copy(dst_hbm.at[pl.ds(off, CH)], didx)
            pltpu.async_copy(g_hbm.at[sidx], rows, sem).wait()
            pltpu.sync_copy(rows, acc_sh.at[didx], add=True)
            if with_deg:
                pltpu.sync_copy(ones_v, deg_sh.at[didx], add=True)

        plsc.subcore_barrier()
        pltpu.sync_copy(acc_sh.at[pl.ds(s * RPT, RPT)],
                        part_hbm.at[c, pl.ds(s * RPT, RPT)])
        if with_deg:
            pltpu.sync_copy(deg_sh.at[pl.ds(s * RPT, RPT)],
                            degp_hbm.at[c, pl.ds(s * RPT, RPT)])

    return pl.kernel(body, out_type=out_type, mesh=mesh, scratch_types=scratch)


_sc_pass1 = _make_sc_segsum(with_deg=True)
_sc_pass2 = _make_sc_segsum(with_deg=False)


def _mm_body(x_ref, w_ref, o_ref):
    o_ref[...] = jnp.dot(x_ref[...], w_ref[...],
                         preferred_element_type=jnp.float32)


_tc_matmul = pl.pallas_call(
    _mm_body,
    grid=(N // ROWS_BLK,),
    in_specs=[
        pl.BlockSpec((ROWS_BLK, D), lambda i: (i, 0)),
        pl.BlockSpec((D, D), lambda i: (0, 0)),
    ],
    out_specs=pl.BlockSpec((ROWS_BLK, D), lambda i: (i, 0)),
    out_shape=jax.ShapeDtypeStruct((N, D), jnp.float32),
)


def _comb_mm_body(p_ref, g_ref, d_ref, b_ref, w_ref, o_ref):
    ssum = p_ref[0] + p_ref[1] + g_ref[...]
    deg = d_ref[0, :, :1] + d_ref[1, :, :1]
    h = ssum * (1.0 / (deg + 1.0)) + b_ref[...]
    h = jnp.maximum(h, 0.0)
    o_ref[...] = jnp.dot(h, w_ref[...], preferred_element_type=jnp.float32)


_tc_comb_matmul = pl.pallas_call(
    _comb_mm_body,
    grid=(N // ROWS_BLK,),
    in_specs=[
        pl.BlockSpec((NC, ROWS_BLK, D), lambda i: (0, i, 0)),
        pl.BlockSpec((ROWS_BLK, D), lambda i: (i, 0)),
        pl.BlockSpec((NC, ROWS_BLK, DEGW), lambda i: (0, i, 0)),
        pl.BlockSpec((1, D), lambda i: (0, 0)),
        pl.BlockSpec((D, D), lambda i: (0, 0)),
    ],
    out_specs=pl.BlockSpec((ROWS_BLK, D), lambda i: (i, 0)),
    out_shape=jax.ShapeDtypeStruct((N, D), jnp.float32),
)


def _final_body(q_ref, g_ref, d_ref, b_ref, o_ref):
    ssum = q_ref[0] + q_ref[1] + g_ref[...]
    deg = d_ref[0, :, :1] + d_ref[1, :, :1]
    o_ref[...] = ssum * (1.0 / (deg + 1.0)) + b_ref[...]


_tc_final = pl.pallas_call(
    _final_body,
    grid=(N // ROWS_BLK,),
    in_specs=[
        pl.BlockSpec((NC, ROWS_BLK, D), lambda i: (0, i, 0)),
        pl.BlockSpec((ROWS_BLK, D), lambda i: (i, 0)),
        pl.BlockSpec((NC, ROWS_BLK, DEGW), lambda i: (0, i, 0)),
        pl.BlockSpec((1, D), lambda i: (0, 0)),
    ],
    out_specs=pl.BlockSpec((ROWS_BLK, D), lambda i: (i, 0)),
    out_shape=jax.ShapeDtypeStruct((N, D), jnp.float32),
)


@jax.jit
def kernel(inputs, edge_index, W1, b1, W2, b2):
    src = edge_index[0]
    dst = edge_index[1]
    zrow = jnp.zeros((RPT, D), jnp.float32)
    zdeg = jnp.zeros((RPT, DEGW), jnp.float32)
    ones = jnp.ones((CH, DEGW), jnp.float32)
    b1r = b1.reshape(1, D)
    b2r = b2.reshape(1, D)

    g1 = _tc_matmul(inputs, W1)
    p1, degp = _sc_pass1(g1, src, dst, zrow, zdeg, ones)
    g2 = _tc_comb_matmul(p1, g1, degp, b1r, W2)
    p2 = _sc_pass2(g2, src, dst, zrow)
    return _tc_final(p2, g2, degp, b2r)


# speedup vs baseline: 5.5860x; 5.5860x over previous
import dataclasses
import functools

import jax
import jax.numpy as jnp
from jax import lax
from jax.experimental import pallas as pl
from jax.experimental.pallas import tpu as pltpu
from jax.experimental.pallas import tpu_sc as plsc

N = 10000
E = 320000
D = 128

NC = 2
NS = 16
NW = NC * NS
EPT = E // NW
CH = 80
NCHUNK = EPT // CH
NP = 10240
RPT = NP // NS
DEGW = 16

ROWS_BLK = 2048
NBLK = NP // ROWS_BLK


@functools.lru_cache(maxsize=None)
def _make_sc_segsum(with_deg: bool):
    mesh = plsc.VectorSubcoreMesh(core_axis_name="c", subcore_axis_name="s")
    out_type = [jax.ShapeDtypeStruct((NC, NP, D), jnp.float32)]
    scratch = [
        pltpu.VMEM((CH,), jnp.int32),
        pltpu.VMEM((CH,), jnp.int32),
        pltpu.VMEM((CH, D), jnp.float32),
        pltpu.VMEM_SHARED((NP, D), jnp.float32),
        pltpu.SemaphoreType.DMA,
    ]
    if with_deg:
        out_type.append(jax.ShapeDtypeStruct((NW, 1, NP), jnp.float32))
        scratch.append(pltpu.VMEM((NP,), jnp.float32))

    def body(*refs):
        if with_deg:
            (g_hbm, src_hbm, dst_hbm, zrow_hbm,
             part_hbm, hist_hbm,
             sidx, didx, rows, acc_sh, sem, hist_v) = refs
        else:
            (g_hbm, src_hbm, dst_hbm, zrow_hbm,
             part_hbm,
             sidx, didx, rows, acc_sh, sem) = refs
        c = lax.axis_index("c")
        s = lax.axis_index("s")
        wid = c * NS + s

        pltpu.sync_copy(zrow_hbm, acc_sh.at[pl.ds(s * RPT, RPT)])
        if with_deg:
            @pl.loop(0, NP // 16)
            def _(i):
                hist_v[pl.ds(i * 16, 16)] = jnp.zeros((16,), jnp.float32)
        plsc.subcore_barrier()

        base = pl.multiple_of(wid * EPT, 8)

        @pl.loop(0, NCHUNK)
        def _(j):
            off = pl.multiple_of(base + j * CH, 8)
            pltpu.sync_copy(src_hbm.at[pl.ds(off, CH)], sidx)
            pltpu.sync_copy(dst_hbm.at[pl.ds(off, CH)], didx)
            pltpu.async_copy(g_hbm.at[sidx], rows, sem).wait()
            pltpu.sync_copy(rows, acc_sh.at[didx], add=True)
            if with_deg:
                ones16 = jnp.ones((16,), jnp.float32)
                for k in range(CH // 16):
                    idx = didx[pl.ds(k * 16, 16)]
                    plsc.addupdate_scatter(hist_v, [idx], ones16)

        plsc.subcore_barrier()
        pltpu.sync_copy(acc_sh.at[pl.ds(s * RPT, RPT)],
                        part_hbm.at[c, pl.ds(s * RPT, RPT)])
        if with_deg:
            pltpu.sync_copy(hist_v, hist_hbm.at[wid, 0])

    cp = pltpu.CompilerParams()
    if "needs_layout_passes" in pltpu.CompilerParams.__dataclass_fields__:
        cp = dataclasses.replace(cp, needs_layout_passes=False)
    return pl.kernel(body, out_type=out_type, mesh=mesh, scratch_types=scratch,
                     compiler_params=cp)


def _mm_body(x_ref, w_ref, o_ref):
    o_ref[...] = jnp.dot(x_ref[...], w_ref[...],
                         preferred_element_type=jnp.float32)


_tc_matmul = pl.pallas_call(
    _mm_body,
    grid=(NBLK,),
    in_specs=[
        pl.BlockSpec((ROWS_BLK, D), lambda i: (i, 0)),
        pl.BlockSpec((D, D), lambda i: (0, 0)),
    ],
    out_specs=pl.BlockSpec((ROWS_BLK, D), lambda i: (i, 0)),
    out_shape=jax.ShapeDtypeStruct((N, D), jnp.float32),
)


def _deg_col(d_ref):
    return lax.dot_general(
        d_ref[:, 0, :], jnp.ones((NW, 1), jnp.float32),
        dimension_numbers=(((0,), (0,)), ((), ())),
        preferred_element_type=jnp.float32)


def _comb_mm_body(p_ref, g_ref, d_ref, b_ref, w_ref, o_ref):
    ssum = p_ref[0] + p_ref[1] + g_ref[...]
    h = ssum * (1.0 / (_deg_col(d_ref) + 1.0)) + b_ref[...]
    h = jnp.maximum(h, 0.0)
    o_ref[...] = jnp.dot(h, w_ref[...], preferred_element_type=jnp.float32)


_tc_comb_matmul = pl.pallas_call(
    _comb_mm_body,
    grid=(NBLK,),
    in_specs=[
        pl.BlockSpec((NC, ROWS_BLK, D), lambda i: (0, i, 0)),
        pl.BlockSpec((ROWS_BLK, D), lambda i: (i, 0)),
        pl.BlockSpec((NW, 1, ROWS_BLK), lambda i: (0, 0, i)),
        pl.BlockSpec((1, D), lambda i: (0, 0)),
        pl.BlockSpec((D, D), lambda i: (0, 0)),
    ],
    out_specs=pl.BlockSpec((ROWS_BLK, D), lambda i: (i, 0)),
    out_shape=jax.ShapeDtypeStruct((N, D), jnp.float32),
)


def _final_body(q_ref, g_ref, d_ref, b_ref, o_ref):
    ssum = q_ref[0] + q_ref[1] + g_ref[...]
    o_ref[...] = ssum * (1.0 / (_deg_col(d_ref) + 1.0)) + b_ref[...]


_tc_final = pl.pallas_call(
    _final_body,
    grid=(NBLK,),
    in_specs=[
        pl.BlockSpec((NC, ROWS_BLK, D), lambda i: (0, i, 0)),
        pl.BlockSpec((ROWS_BLK, D), lambda i: (i, 0)),
        pl.BlockSpec((NW, 1, ROWS_BLK), lambda i: (0, 0, i)),
        pl.BlockSpec((1, D), lambda i: (0, 0)),
    ],
    out_specs=pl.BlockSpec((ROWS_BLK, D), lambda i: (i, 0)),
    out_shape=jax.ShapeDtypeStruct((N, D), jnp.float32),
)


@jax.jit
def kernel(inputs, edge_index, W1, b1, W2, b2):
    src = edge_index[0]
    dst = edge_index[1]
    zrow = jnp.zeros((RPT, D), jnp.float32)
    b1r = b1.reshape(1, D)
    b2r = b2.reshape(1, D)

    g1 = _tc_matmul(inputs, W1)
    p1, hist = _make_sc_segsum(True)(g1, src, dst, zrow)
    g2 = _tc_comb_matmul(p1, g1, hist, b1r, W2)
    (p2,) = _make_sc_segsum(False)(g2, src, dst, zrow)
    return _tc_final(p2, g2, hist, b2r)


# speedup vs baseline: 9.5947x; 1.7176x over previous
import dataclasses
import functools

import jax
import jax.numpy as jnp
from jax import lax
from jax.experimental import pallas as pl
from jax.experimental.pallas import tpu as pltpu
from jax.experimental.pallas import tpu_sc as plsc

N = 10000
E = 320000
D = 128

NC = 2
NS = 16
NW = NC * NS
EPT = E // NW
CH = 80
NCHUNK = EPT // CH
NBUF = 3
NGRP = NCHUNK // NBUF
NP = 10240
RPT = NP // NS
DEGW = 16

ROWS_BLK = 2048
NBLK = NP // ROWS_BLK


@functools.lru_cache(maxsize=None)
def _make_sc_segsum(with_deg: bool):
    mesh = plsc.VectorSubcoreMesh(core_axis_name="c", subcore_axis_name="s")
    out_type = [jax.ShapeDtypeStruct((NC, NP, D), jnp.float32)]
    scratch = (
        [pltpu.VMEM((CH,), jnp.int32) for _ in range(NBUF)]
        + [pltpu.VMEM((CH,), jnp.int32) for _ in range(NBUF)]
        + [pltpu.VMEM((CH, D), jnp.float32) for _ in range(NBUF)]
        + [pltpu.VMEM_SHARED((NP, D), jnp.float32)]
        + [pltpu.SemaphoreType.DMA for _ in range(NBUF + 2)]
    )
    if with_deg:
        out_type.append(jax.ShapeDtypeStruct((NW, 1, NP), jnp.float32))
        scratch.append(pltpu.VMEM((NP,), jnp.float32))

    def body(*refs):
        n_in = 4
        n_out = 2 if with_deg else 1
        refs_in = refs[:n_in]
        refs_out = refs[n_in:n_in + n_out]
        sc = list(refs[n_in + n_out:])
        g_hbm, src_hbm, dst_hbm, zrow_hbm = refs_in
        part_hbm = refs_out[0]
        sidx = sc[:NBUF]
        didx = sc[NBUF:2 * NBUF]
        rows = sc[2 * NBUF:3 * NBUF]
        acc_sh = sc[3 * NBUF]
        sem_i = sc[3 * NBUF + 1]
        sem_g = sc[3 * NBUF + 2:3 * NBUF + 2 + NBUF]
        sem_s = sc[3 * NBUF + 2 + NBUF]
        if with_deg:
            hist_hbm = refs_out[1]
            hist_v = sc[3 * NBUF + 3 + NBUF]
        c = lax.axis_index("c")
        s = lax.axis_index("s")
        wid = c * NS + s

        pltpu.sync_copy(zrow_hbm, acc_sh.at[pl.ds(s * RPT, RPT)])
        if with_deg:
            @pl.loop(0, NP // 16)
            def _(i):
                hist_v[pl.ds(i * 16, 16)] = jnp.zeros((16,), jnp.float32)
        plsc.subcore_barrier()

        base = pl.multiple_of(wid * EPT, 8)

        @pl.loop(0, NGRP)
        def _(g):
            off0 = base + g * (NBUF * CH)
            ih = []
            for b in range(NBUF):
                off = pl.multiple_of(off0 + b * CH, 8)
                h1 = pltpu.async_copy(src_hbm.at[pl.ds(off, CH)], sidx[b],
                                      sem_i)
                h2 = pltpu.async_copy(dst_hbm.at[pl.ds(off, CH)], didx[b],
                                      sem_i)
                ih.append((h1, h2))
            gh = []
            for b in range(NBUF):
                ih[b][0].wait()
                ih[b][1].wait()
                gh.append(pltpu.async_copy(g_hbm.at[sidx[b]], rows[b],
                                           sem_g[b]))
            sh = []
            for b in range(NBUF):
                gh[b].wait()
                if with_deg:
                    ones16 = jnp.ones((16,), jnp.float32)
                    for k in range(CH // 16):
                        idx = didx[b][pl.ds(k * 16, 16)]
                        plsc.addupdate_scatter(hist_v, [idx], ones16)
                sh.append(pltpu.async_copy(rows[b], acc_sh.at[didx[b]],
                                           sem_s, add=True))
            for h in sh:
                h.wait()

        for j in range(NGRP * NBUF, NCHUNK):
            off = pl.multiple_of(base + j * CH, 8)
            pltpu.sync_copy(src_hbm.at[pl.ds(off, CH)], sidx[0])
            pltpu.sync_copy(dst_hbm.at[pl.ds(off, CH)], didx[0])
            pltpu.async_copy(g_hbm.at[sidx[0]], rows[0], sem_g[0]).wait()
            if with_deg:
                ones16 = jnp.ones((16,), jnp.float32)
                for k in range(CH // 16):
                    idx = didx[0][pl.ds(k * 16, 16)]
                    plsc.addupdate_scatter(hist_v, [idx], ones16)
            pltpu.sync_copy(rows[0], acc_sh.at[didx[0]], add=True)

        plsc.subcore_barrier()
        pltpu.sync_copy(acc_sh.at[pl.ds(s * RPT, RPT)],
                        part_hbm.at[c, pl.ds(s * RPT, RPT)])
        if with_deg:
            pltpu.sync_copy(hist_v, hist_hbm.at[wid, 0])

    cp = pltpu.CompilerParams()
    if "needs_layout_passes" in pltpu.CompilerParams.__dataclass_fields__:
        cp = dataclasses.replace(cp, needs_layout_passes=False)
    return pl.kernel(body, out_type=out_type, mesh=mesh, scratch_types=scratch,
                     compiler_params=cp)


def _mm_body(x_ref, w_ref, o_ref):
    o_ref[...] = jnp.dot(x_ref[...], w_ref[...],
                         preferred_element_type=jnp.float32)


_tc_matmul = pl.pallas_call(
    _mm_body,
    grid=(NBLK,),
    in_specs=[
        pl.BlockSpec((ROWS_BLK, D), lambda i: (i, 0)),
        pl.BlockSpec((D, D), lambda i: (0, 0)),
    ],
    out_specs=pl.BlockSpec((ROWS_BLK, D), lambda i: (i, 0)),
    out_shape=jax.ShapeDtypeStruct((N, D), jnp.float32),
)


def _deg_col(d_ref):
    return lax.dot_general(
        d_ref[:, 0, :], jnp.ones((NW, 1), jnp.float32),
        dimension_numbers=(((0,), (0,)), ((), ())),
        preferred_element_type=jnp.float32)


def _comb_mm_body(p_ref, g_ref, d_ref, b_ref, w_ref, o_ref):
    ssum = p_ref[0] + p_ref[1] + g_ref[...]
    h = ssum * (1.0 / (_deg_col(d_ref) + 1.0)) + b_ref[...]
    h = jnp.maximum(h, 0.0)
    o_ref[...] = jnp.dot(h, w_ref[...], preferred_element_type=jnp.float32)


_tc_comb_matmul = pl.pallas_call(
    _comb_mm_body,
    grid=(NBLK,),
    in_specs=[
        pl.BlockSpec((NC, ROWS_BLK, D), lambda i: (0, i, 0)),
        pl.BlockSpec((ROWS_BLK, D), lambda i: (i, 0)),
        pl.BlockSpec((NW, 1, ROWS_BLK), lambda i: (0, 0, i)),
        pl.BlockSpec((1, D), lambda i: (0, 0)),
        pl.BlockSpec((D, D), lambda i: (0, 0)),
    ],
    out_specs=pl.BlockSpec((ROWS_BLK, D), lambda i: (i, 0)),
    out_shape=jax.ShapeDtypeStruct((N, D), jnp.float32),
)


def _final_body(q_ref, g_ref, d_ref, b_ref, o_ref):
    ssum = q_ref[0] + q_ref[1] + g_ref[...]
    o_ref[...] = ssum * (1.0 / (_deg_col(d_ref) + 1.0)) + b_ref[...]


_tc_final = pl.pallas_call(
    _final_body,
    grid=(NBLK,),
    in_specs=[
        pl.BlockSpec((NC, ROWS_BLK, D), lambda i: (0, i, 0)),
        pl.BlockSpec((ROWS_BLK, D), lambda i: (i, 0)),
        pl.BlockSpec((NW, 1, ROWS_BLK), lambda i: (0, 0, i)),
        pl.BlockSpec((1, D), lambda i: (0, 0)),
    ],
    out_specs=pl.BlockSpec((ROWS_BLK, D), lambda i: (i, 0)),
    out_shape=jax.ShapeDtypeStruct((N, D), jnp.float32),
)


@jax.jit
def kernel(inputs, edge_index, W1, b1, W2, b2):
    src = edge_index[0]
    dst = edge_index[1]
    zrow = jnp.zeros((RPT, D), jnp.float32)
    b1r = b1.reshape(1, D)
    b2r = b2.reshape(1, D)

    g1 = _tc_matmul(inputs, W1)
    p1, hist = _make_sc_segsum(True)(g1, src, dst, zrow)
    g2 = _tc_comb_matmul(p1, g1, hist, b1r, W2)
    (p2,) = _make_sc_segsum(False)(g2, src, dst, zrow)
    return _tc_final(p2, g2, hist, b2r)


# speedup vs baseline: 10.3160x; 1.0752x over previous
import dataclasses
import functools

import jax
import jax.numpy as jnp
from jax import lax
from jax.experimental import pallas as pl
from jax.experimental.pallas import tpu as pltpu
from jax.experimental.pallas import tpu_sc as plsc

N = 10000
E = 320000
D = 128
DH = D // 2

NC = 2
NS = 16
EPT = E // NS
CH = 80
NCHUNK = EPT // CH
SETS = 2
NBUF = 5
GRP = SETS * NBUF
NGRP = NCHUNK // GRP
NP = 10240
RPT = NP // NS

ROWS_BLK = 2048
NBLK = NP // ROWS_BLK


@functools.lru_cache(maxsize=None)
def _make_sc_segsum(with_deg: bool):
    mesh = plsc.VectorSubcoreMesh(core_axis_name="c", subcore_axis_name="s")
    out_type = [jax.ShapeDtypeStruct((NC, NP, DH), jnp.float32)]
    scratch = (
        [pltpu.VMEM((CH,), jnp.int32) for _ in range(GRP)]
        + [pltpu.VMEM((CH,), jnp.int32) for _ in range(GRP)]
        + [pltpu.VMEM((CH, DH), jnp.float32) for _ in range(GRP)]
        + [pltpu.VMEM_SHARED((NP, DH), jnp.float32)]
        + [pltpu.SemaphoreType.DMA for _ in range(SETS)]
        + [pltpu.SemaphoreType.DMA for _ in range(GRP)]
        + [pltpu.SemaphoreType.DMA for _ in range(SETS)]
    )
    if with_deg:
        out_type.append(jax.ShapeDtypeStruct((NS, 1, NP), jnp.float32))
        scratch.append(pltpu.VMEM((NP,), jnp.float32))

    def body(*refs):
        n_in = 5
        n_out = 2 if with_deg else 1
        (glo_hbm, ghi_hbm, src_hbm, dst_hbm, zrow_hbm) = refs[:n_in]
        part_hbm = refs[n_in]
        sc = list(refs[n_in + n_out:])
        sidx = sc[:GRP]
        didx = sc[GRP:2 * GRP]
        rows = sc[2 * GRP:3 * GRP]
        acc_sh = sc[3 * GRP]
        sem_i = sc[3 * GRP + 1:3 * GRP + 1 + SETS]
        sem_g = sc[3 * GRP + 1 + SETS:3 * GRP + 1 + SETS + GRP]
        sem_s = sc[3 * GRP + 1 + SETS + GRP:3 * GRP + 1 + 2 * SETS + GRP]
        if with_deg:
            hist_hbm = refs[n_in + 1]
            hist_v = sc[3 * GRP + 1 + 2 * SETS + GRP]
        c = lax.axis_index("c")
        s = lax.axis_index("s")

        pltpu.sync_copy(zrow_hbm, acc_sh.at[pl.ds(s * RPT, RPT)])
        if with_deg:
            @pl.loop(0, NP // 16)
            def _(i):
                hist_v[pl.ds(i * 16, 16)] = jnp.zeros((16,), jnp.float32)
        plsc.subcore_barrier()

        base = pl.multiple_of(s * EPT, 8)

        def drain_set(sidx_set):
            for b in sidx_set:
                pltpu.make_async_copy(rows[b], acc_sh.at[didx[b]],
                                      sem_s[0 if b < NBUF else 1]).wait()

        @pl.loop(0, NGRP)
        def _(g):
            for S in range(SETS):
                bs = list(range(S * NBUF, (S + 1) * NBUF))

                @pl.when(g > 0)
                def _():
                    drain_set(bs)

                off0 = base + (g * GRP + S * NBUF) * CH
                ih = []
                for k, b in enumerate(bs):
                    off = pl.multiple_of(off0 + k * CH, 8)
                    h1 = pltpu.async_copy(src_hbm.at[pl.ds(off, CH)],
                                          sidx[b], sem_i[S])
                    h2 = pltpu.async_copy(dst_hbm.at[pl.ds(off, CH)],
                                          didx[b], sem_i[S])
                    ih.append((h1, h2))
                for k, b in enumerate(bs):
                    ih[k][0].wait()
                    ih[k][1].wait()

                    @pl.when(c == 0)
                    def _():
                        pltpu.async_copy(glo_hbm.at[sidx[b]], rows[b],
                                         sem_g[b])

                    @pl.when(c == 1)
                    def _():
                        pltpu.async_copy(ghi_hbm.at[sidx[b]], rows[b],
                                         sem_g[b])
                for b in bs:
                    pltpu.make_async_copy(glo_hbm.at[sidx[b]], rows[b],
                                          sem_g[b]).wait()
                    if with_deg:
                        @pl.when(c == 0)
                        def _():
                            ones16 = jnp.ones((16,), jnp.float32)
                            for k in range(CH // 16):
                                idx = didx[b][pl.ds(k * 16, 16)]
                                plsc.addupdate_scatter(hist_v, [idx], ones16)
                    pltpu.async_copy(rows[b], acc_sh.at[didx[b]],
                                     sem_s[S], add=True)

        drain_set(list(range(GRP)))

        plsc.subcore_barrier()
        pltpu.sync_copy(acc_sh.at[pl.ds(s * RPT, RPT)],
                        part_hbm.at[c, pl.ds(s * RPT, RPT)])
        if with_deg:
            @pl.when(c == 0)
            def _():
                pltpu.sync_copy(hist_v, hist_hbm.at[s, 0])

    cp = pltpu.CompilerParams()
    if "needs_layout_passes" in pltpu.CompilerParams.__dataclass_fields__:
        cp = dataclasses.replace(cp, needs_layout_passes=False)
    if "use_tc_tiling_on_sc" in pltpu.CompilerParams.__dataclass_fields__:
        cp = dataclasses.replace(cp, use_tc_tiling_on_sc=False)
    return pl.kernel(body, out_type=out_type, mesh=mesh, scratch_types=scratch,
                     compiler_params=cp)


def _mm_body(x_ref, w_ref, o0_ref, o1_ref):
    h = jnp.dot(x_ref[...], w_ref[...], preferred_element_type=jnp.float32)
    o0_ref[...] = h[:, :DH]
    o1_ref[...] = h[:, DH:]


_tc_matmul = pl.pallas_call(
    _mm_body,
    grid=(NBLK,),
    in_specs=[
        pl.BlockSpec((ROWS_BLK, D), lambda i: (i, 0)),
        pl.BlockSpec((D, D), lambda i: (0, 0)),
    ],
    out_specs=[pl.BlockSpec((ROWS_BLK, DH), lambda i: (i, 0))] * 2,
    out_shape=[jax.ShapeDtypeStruct((N, DH), jnp.float32)] * 2,
)


def _deg_col(d_ref):
    return lax.dot_general(
        d_ref[:, 0, :], jnp.ones((NS, 1), jnp.float32),
        dimension_numbers=(((0,), (0,)), ((), ())),
        preferred_element_type=jnp.float32)


def _halves(p_ref, glo_ref, ghi_ref, d_ref, b_ref):
    rinv = 1.0 / (_deg_col(d_ref) + 1.0)
    lo = (p_ref[0] + glo_ref[...]) * rinv + b_ref[:, :DH]
    hi = (p_ref[1] + ghi_ref[...]) * rinv + b_ref[:, DH:]
    return lo, hi


def _comb_mm_body(p_ref, glo_ref, ghi_ref, d_ref, b_ref, w_ref,
                  o0_ref, o1_ref):
    lo, hi = _halves(p_ref, glo_ref, ghi_ref, d_ref, b_ref)
    h = jnp.maximum(jnp.concatenate([lo, hi], axis=1), 0.0)
    g2 = jnp.dot(h, w_ref[...], preferred_element_type=jnp.float32)
    o0_ref[...] = g2[:, :DH]
    o1_ref[...] = g2[:, DH:]


_in_specs_comb = [
    pl.BlockSpec((NC, ROWS_BLK, DH), lambda i: (0, i, 0)),
    pl.BlockSpec((ROWS_BLK, DH), lambda i: (i, 0)),
    pl.BlockSpec((ROWS_BLK, DH), lambda i: (i, 0)),
    pl.BlockSpec((NS, 1, ROWS_BLK), lambda i: (0, 0, i)),
    pl.BlockSpec((1, D), lambda i: (0, 0)),
]

_tc_comb_matmul = pl.pallas_call(
    _comb_mm_body,
    grid=(NBLK,),
    in_specs=_in_specs_comb + [pl.BlockSpec((D, D), lambda i: (0, 0))],
    out_specs=[pl.BlockSpec((ROWS_BLK, DH), lambda i: (i, 0))] * 2,
    out_shape=[jax.ShapeDtypeStruct((N, DH), jnp.float32)] * 2,
)


def _final_body(q_ref, glo_ref, ghi_ref, d_ref, b_ref, o_ref):
    lo, hi = _halves(q_ref, glo_ref, ghi_ref, d_ref, b_ref)
    o_ref[...] = jnp.concatenate([lo, hi], axis=1)


_tc_final = pl.pallas_call(
    _final_body,
    grid=(NBLK,),
    in_specs=_in_specs_comb,
    out_specs=pl.BlockSpec((ROWS_BLK, D), lambda i: (i, 0)),
    out_shape=jax.ShapeDtypeStruct((N, D), jnp.float32),
)


@jax.jit
def kernel(inputs, edge_index, W1, b1, W2, b2):
    src = edge_index[0]
    dst = edge_index[1]
    zrow = jnp.zeros((RPT, DH), jnp.float32)
    b1r = b1.reshape(1, D)
    b2r = b2.reshape(1, D)

    g1lo, g1hi = _tc_matmul(inputs, W1)
    p1, hist = _make_sc_segsum(True)(g1lo, g1hi, src, dst, zrow)
    g2lo, g2hi = _tc_comb_matmul(p1, g1lo, g1hi, hist, b1r, W2)
    (p2,) = _make_sc_segsum(False)(g2lo, g2hi, src, dst, zrow)
    return _tc_final(p2, g2lo, g2hi, hist, b2r)
